# baseline (speedup 1.0000x reference)
import functools

import jax
import jax.numpy as jnp
from jax import lax
from jax.experimental import pallas as pl
from jax.experimental.pallas import tpu as pltpu
from jax.experimental.pallas import tpu_sc as plsc

B, R, C = 16, 512, 512
NC, NS, L = 2, 16, 16
NW = NC * NS
ROWS_PER_TILE = R // 2
CHUNK_ROWS = 32
NCHUNK = ROWS_PER_TILE // CHUNK_ROWS
GROUPS = CHUNK_ROWS * C // (4 * L)
MU0 = 0.8

_mesh = plsc.VectorSubcoreMesh(
    core_axis_name="c", subcore_axis_name="s", num_cores=NC, num_subcores=NS
)


@functools.partial(
    pl.kernel,
    compiler_params=pltpu.CompilerParams(needs_layout_passes=False),
    out_type=jax.ShapeDtypeStruct((NW, 4, L), jnp.float32),
    mesh=_mesh,
    scratch_types=[
        pltpu.VMEM((CHUNK_ROWS, C), jnp.float32),
        pltpu.VMEM((CHUNK_ROWS, C), jnp.float32),
        pltpu.VMEM((CHUNK_ROWS, C), jnp.float32),
        pltpu.VMEM((CHUNK_ROWS, C), jnp.float32),
        pltpu.VMEM((CHUNK_ROWS, C), jnp.float32),
        pltpu.VMEM((CHUNK_ROWS, C), jnp.float32),
        pltpu.VMEM((8, L), jnp.float32),
        pltpu.VMEM((4, L), jnp.float32),
        pltpu.SemaphoreType.DMA,
        pltpu.SemaphoreType.DMA,
        pltpu.SemaphoreType.DMA,
    ],
)
def _partials(a_hbm, b_hbm, m_hbm, sc_hbm, part_hbm,
              a0, a1, b0, b1, m0, m1, scv, stage, sem0, sem1, scsem):
    wid = lax.axis_index("s") * NC + lax.axis_index("c")
    bidx = wid >> 1
    rbase = (wid & 1) * ROWS_PER_TILE
    abufs, bbufs, mbufs, sems = (a0, a1), (b0, b1), (m0, m1), (sem0, sem1)

    pltpu.async_copy(sc_hbm, scv, scsem)

    def start(k):
        cur = k % 2
        rows = pl.ds(rbase + k * CHUNK_ROWS, CHUNK_ROWS)
        return (
            pltpu.async_copy(a_hbm.at[bidx, rows, :], abufs[cur], sems[cur]),
            pltpu.async_copy(b_hbm.at[bidx, rows, :], bbufs[cur], sems[cur]),
            pltpu.async_copy(m_hbm.at[bidx, rows, :], mbufs[cur], sems[cur]),
        )

    handles = [None] * NCHUNK
    handles[0] = start(0)

    pltpu.make_async_copy(sc_hbm, scv, scsem).wait()
    r_sa = scv[0, :]
    r_o1 = scv[1, :]
    r_s2 = scv[2, :]
    r_sb = scv[3, :]
    r_ob = scv[4, :]

    n = jnp.zeros((L,), jnp.float32)
    s = jnp.zeros((L,), jnp.float32)
    q = jnp.zeros((L,), jnp.float32)

    for k in range(NCHUNK):
        cur = k % 2
        if k + 1 < NCHUNK:
            handles[k + 1] = start(k + 1)
        for h in handles[k]:
            h.wait()
        av, bv, mv = abufs[cur], bbufs[cur], mbufs[cur]

        def g_body(g, carry, av=av, bv=bv, mv=mv):
            n, s, q = carry
            row = g >> 3
            colbase = (g & 7) * 64
            for j in range(4):
                col = pl.ds(colbase + 16 * j, L)
                va = av[row, col]
                vb = bv[row, col]
                m = mv[row, col]
                bb = vb * r_sb + r_ob
                num = (va * r_sa + r_o1) + vb * r_s2
                t = num / bb
                dm = m * t
                n = n + m
                s = s + dm
                q = q + dm * t
            return (n, s, q)

        n, s, q = lax.fori_loop(0, GROUPS, g_body, (n, s, q))

    stage[0, :] = n
    stage[1, :] = s
    stage[2, :] = q
    stage[3, :] = jnp.zeros((L,), jnp.float32)
    pltpu.sync_copy(stage, part_hbm.at[wid])


@functools.partial(
    pl.kernel,
    compiler_params=pltpu.CompilerParams(needs_layout_passes=False),
    out_type=jax.ShapeDtypeStruct((L,), jnp.float32),
    mesh=_mesh,
    scratch_types=[
        pltpu.VMEM((NW, 4, L), jnp.float32),
        pltpu.VMEM((L,), jnp.float32),
    ],
)
def _finalize(part_hbm, out_hbm, pv, ov):
    wid = lax.axis_index("s") * NC + lax.axis_index("c")

    @pl.when(wid == 0)
    def _():
        pltpu.sync_copy(part_hbm, pv)
        n = jnp.zeros((L,), jnp.float32)
        s = jnp.zeros((L,), jnp.float32)
        q = jnp.zeros((L,), jnp.float32)
        for t in range(NW):
            n = n + pv[t, 0, :]
            s = s + pv[t, 1, :]
            q = q + pv[t, 2, :]
        ns = jnp.broadcast_to(jnp.sum(n), (L,))
        ss = jnp.broadcast_to(jnp.sum(s), (L,))
        qs = jnp.broadcast_to(jnp.sum(q), (L,))
        mean_sh = ss / ns
        mean = mean_sh + MU0
        var = (qs - ss * mean_sh) / (ns - 1.0)
        ov[:] = var / mean
        pltpu.sync_copy(ov, out_hbm)


def kernel(a, b, ts, mina, maxa, minb, maxb):
    m = ts.astype(jnp.float32)
    sa = maxa - mina
    sb = maxb - minb
    rows = jnp.stack([
        jnp.broadcast_to(sa, (L,)),
        jnp.broadcast_to(mina - MU0 * minb, (L,)),
        jnp.broadcast_to(-MU0 * sb, (L,)),
        jnp.broadcast_to(sb, (L,)),
        jnp.broadcast_to(minb, (L,)),
        jnp.zeros((L,), jnp.float32),
        jnp.zeros((L,), jnp.float32),
        jnp.zeros((L,), jnp.float32),
    ])
    parts = _partials(a, b, m, rows)
    out = _finalize(parts)
    return out[0]
